# Initial kernel scaffold; baseline (speedup 1.0000x reference)
import jax
import jax.numpy as jnp
from jax import lax
from jax.experimental import pallas as pl
from jax.experimental.pallas import tpu as pltpu
from jax.experimental.pallas import tpu_sc as plsc

N_NODES = 10000
D = 128
NUM_CORES = 2
NUM_SUBCORES = 16
NW = NUM_CORES * NUM_SUBCORES
CH = 128


def _sc_mesh():
    return plsc.VectorSubcoreMesh(core_axis_name="c", subcore_axis_name="s")


def _preproject(x, w_sr, w_a):
    rb = 2000

    def body(x_ref, wsr_ref, wa_ref, p_ref, xa_ref):
        xb = x_ref[...]
        p_ref[...] = jnp.dot(xb, wsr_ref[...], preferred_element_type=jnp.float32)
        xa_ref[...] = jnp.dot(xb, wa_ref[...], preferred_element_type=jnp.float32)

    p, xa = pl.pallas_call(
        body,
        grid=(N_NODES // rb,),
        in_specs=[
            pl.BlockSpec((rb, D), lambda i: (i, 0)),
            pl.BlockSpec((D, 4 * D), lambda i: (0, 0)),
            pl.BlockSpec((D, D), lambda i: (0, 0)),
        ],
        out_specs=[
            pl.BlockSpec((rb, 4 * D), lambda i: (i, 0)),
            pl.BlockSpec((rb, D), lambda i: (i, 0)),
        ],
        out_shape=[
            jax.ShapeDtypeStruct((N_NODES, 4 * D), jnp.float32),
            jax.ShapeDtypeStruct((N_NODES, D), jnp.float32),
        ],
    )(x, w_sr, w_a)
    return p, xa


def _gather_pair(table_s, table_r, sidx, ridx, n_edges):
    per_w = n_edges // NW
    n_full, tail = divmod(per_w, CH)

    def body(ts_h, tr_h, si_h, ri_h, os_h, or_h,
             si_v, ri_v, bs_v, br_v, sit_v, rit_v, sem1, sem2):
        wid = lax.axis_index("s") * NUM_CORES + lax.axis_index("c")
        base = wid * per_w

        def chunk(i, carry):
            off = base + i * CH
            pltpu.sync_copy(si_h.at[pl.ds(off, CH)], si_v)
            pltpu.sync_copy(ri_h.at[pl.ds(off, CH)], ri_v)
            c1 = pltpu.async_copy(ts_h.at[si_v], bs_v, sem1)
            c2 = pltpu.async_copy(tr_h.at[ri_v], br_v, sem2)
            c1.wait()
            c2.wait()
            pltpu.sync_copy(bs_v, os_h.at[pl.ds(off, CH)])
            pltpu.sync_copy(br_v, or_h.at[pl.ds(off, CH)])
            return carry

        lax.fori_loop(0, n_full, chunk, None)
        if tail:
            off = base + n_full * CH
            pltpu.sync_copy(si_h.at[pl.ds(off, tail)], sit_v)
            pltpu.sync_copy(ri_h.at[pl.ds(off, tail)], rit_v)
            c1 = pltpu.async_copy(ts_h.at[sit_v], bs_v.at[pl.ds(0, tail)], sem1)
            c2 = pltpu.async_copy(tr_h.at[rit_v], br_v.at[pl.ds(0, tail)], sem2)
            c1.wait()
            c2.wait()
            pltpu.sync_copy(bs_v.at[pl.ds(0, tail)], os_h.at[pl.ds(off, tail)])
            pltpu.sync_copy(br_v.at[pl.ds(0, tail)], or_h.at[pl.ds(off, tail)])

    kern = pl.kernel(
        body,
        out_type=(
            jax.ShapeDtypeStruct((n_edges, D), jnp.float32),
            jax.ShapeDtypeStruct((n_edges, D), jnp.float32),
        ),
        mesh=_sc_mesh(),
        scratch_types=[
            pltpu.VMEM((CH,), jnp.int32),
            pltpu.VMEM((CH,), jnp.int32),
            pltpu.VMEM((CH, D), jnp.float32),
            pltpu.VMEM((CH, D), jnp.float32),
            pltpu.VMEM((max(tail, 8),), jnp.int32),
            pltpu.VMEM((max(tail, 8),), jnp.int32),
            pltpu.SemaphoreType.DMA,
            pltpu.SemaphoreType.DMA,
        ],
    )
    return kern(table_s, table_r, sidx, ridx)


def _edge_mlp(gs, gr, ef, w1c, b1, w2, b2, w3, b3, g, be, wproj, n_edges, eb):
    def body(gs_ref, gr_ref, ef_ref, w1c_ref, b1_ref, w2_ref, b2_ref,
             w3_ref, b3_ref, g_ref, be_ref, wp_ref, oe_ref, op_ref):
        ef_b = ef_ref[...]
        h = gs_ref[...] + gr_ref[...] + b1_ref[...] + jnp.dot(
            ef_b, w1c_ref[...], preferred_element_type=jnp.float32)
        h = jnp.maximum(h, 0.0)
        h = jnp.maximum(
            jnp.dot(h, w2_ref[...], preferred_element_type=jnp.float32)
            + b2_ref[...], 0.0)
        h = jnp.dot(h, w3_ref[...], preferred_element_type=jnp.float32) + b3_ref[...]
        mu = jnp.mean(h, axis=-1, keepdims=True)
        var = jnp.mean(jnp.square(h - mu), axis=-1, keepdims=True)
        y = (h - mu) / jnp.sqrt(var + 1e-5) * g_ref[...] + be_ref[...]
        oe_ref[...] = y + ef_b
        op_ref[...] = jnp.dot(y, wp_ref[...], preferred_element_type=jnp.float32)

    mat = lambda i: (0, 0)
    new_e, proj = pl.pallas_call(
        body,
        grid=(n_edges // eb,),
        in_specs=[
            pl.BlockSpec((eb, D), lambda i: (i, 0)),
            pl.BlockSpec((eb, D), lambda i: (i, 0)),
            pl.BlockSpec((eb, D), lambda i: (i, 0)),
            pl.BlockSpec((D, D), mat),
            pl.BlockSpec((1, D), mat),
            pl.BlockSpec((D, D), mat),
            pl.BlockSpec((1, D), mat),
            pl.BlockSpec((D, D), mat),
            pl.BlockSpec((1, D), mat),
            pl.BlockSpec((1, D), mat),
            pl.BlockSpec((1, D), mat),
            pl.BlockSpec((D, D), mat),
        ],
        out_specs=[
            pl.BlockSpec((eb, D), lambda i: (i, 0)),
            pl.BlockSpec((eb, D), lambda i: (i, 0)),
        ],
        out_shape=[
            jax.ShapeDtypeStruct((n_edges, D), jnp.float32),
            jax.ShapeDtypeStruct((n_edges, D), jnp.float32),
        ],
    )(gs, gr, ef, w1c, b1, w2, b2, w3, b3, g, be, wproj)
    return new_e, proj


def _scatter_partials(pm, im, pw, iw, em, ew):
    em_h = em // NUM_CORES
    ew_h = ew // NUM_CORES
    em_t = em_h // NUM_SUBCORES
    ew_t = ew_h // NUM_SUBCORES
    m_full, m_tail = divmod(em_t, CH)
    w_full, w_tail = divmod(ew_t, CH)
    rows_t = N_NODES // NUM_SUBCORES

    def body(pm_h, im_h, pw_h, iw_h, zero_h, out_h,
             buf_v, idx_v, mt_v, wt_v, acc_s, sem):
        cid = lax.axis_index("c")
        sid = lax.axis_index("s")

        r0 = sid * rows_t
        pltpu.sync_copy(zero_h.at[pl.ds(r0, rows_t)], acc_s.at[pl.ds(r0, rows_t)])
        plsc.subcore_barrier()

        def scatter_chunks(p_h, i_h, base, nfull, tail, tidx_v):
            def chunk(i, carry):
                off = base + i * CH
                pltpu.sync_copy(i_h.at[pl.ds(off, CH)], idx_v)
                pltpu.sync_copy(p_h.at[pl.ds(off, CH)], buf_v)
                pltpu.sync_copy(buf_v, acc_s.at[idx_v], add=True)
                return carry

            lax.fori_loop(0, nfull, chunk, None)
            if tail:
                off = base + nfull * CH
                pltpu.sync_copy(i_h.at[pl.ds(off, tail)], tidx_v)
                pltpu.sync_copy(p_h.at[pl.ds(off, tail)], buf_v.at[pl.ds(0, tail)])
                pltpu.sync_copy(buf_v.at[pl.ds(0, tail)], acc_s.at[tidx_v], add=True)

        scatter_chunks(pm_h, im_h, cid * em_h + sid * em_t, m_full, m_tail, mt_v)
        scatter_chunks(pw_h, iw_h, cid * ew_h + sid * ew_t, w_full, w_tail, wt_v)
        plsc.subcore_barrier()

        pltpu.sync_copy(acc_s.at[pl.ds(r0, rows_t)],
                        out_h.at[cid].at[pl.ds(r0, rows_t)])

    kern = pl.kernel(
        body,
        out_type=jax.ShapeDtypeStruct((NUM_CORES, N_NODES, D), jnp.float32),
        mesh=_sc_mesh(),
        scratch_types=[
            pltpu.VMEM((CH, D), jnp.float32),
            pltpu.VMEM((CH,), jnp.int32),
            pltpu.VMEM((max(m_tail, 8),), jnp.int32),
            pltpu.VMEM((max(w_tail, 8),), jnp.int32),
            pltpu.VMEM_SHARED((N_NODES, D), jnp.float32),
            pltpu.SemaphoreType.DMA,
        ],
    )
    zero = jnp.zeros((N_NODES, D), jnp.float32)
    return kern(pm, im, pw, iw, zero)


def _node_mlp(x, xa, parts, b1, w2, b2, w3, b3, g, be):
    rb = 2000

    def body(x_ref, xa_ref, p0_ref, p1_ref, b1_ref, w2_ref, b2_ref,
             w3_ref, b3_ref, g_ref, be_ref, o_ref):
        h = xa_ref[...] + p0_ref[0] + p1_ref[0] + b1_ref[...]
        h = jnp.maximum(h, 0.0)
        h = jnp.maximum(
            jnp.dot(h, w2_ref[...], preferred_element_type=jnp.float32)
            + b2_ref[...], 0.0)
        h = jnp.dot(h, w3_ref[...], preferred_element_type=jnp.float32) + b3_ref[...]
        mu = jnp.mean(h, axis=-1, keepdims=True)
        var = jnp.mean(jnp.square(h - mu), axis=-1, keepdims=True)
        y = (h - mu) / jnp.sqrt(var + 1e-5) * g_ref[...] + be_ref[...]
        o_ref[...] = y + x_ref[...]

    mat = lambda i: (0, 0)
    return pl.pallas_call(
        body,
        grid=(N_NODES // rb,),
        in_specs=[
            pl.BlockSpec((rb, D), lambda i: (i, 0)),
            pl.BlockSpec((rb, D), lambda i: (i, 0)),
            pl.BlockSpec((1, rb, D), lambda i: (0, i, 0)),
            pl.BlockSpec((1, rb, D), lambda i: (1, i, 0)),
            pl.BlockSpec((1, D), mat),
            pl.BlockSpec((D, D), mat),
            pl.BlockSpec((1, D), mat),
            pl.BlockSpec((D, D), mat),
            pl.BlockSpec((1, D), mat),
            pl.BlockSpec((1, D), mat),
            pl.BlockSpec((1, D), mat),
        ],
        out_specs=pl.BlockSpec((rb, D), lambda i: (i, 0)),
        out_shape=jax.ShapeDtypeStruct((N_NODES, D), jnp.float32),
    )(x, xa, parts, parts, b1, w2, b2, w3, b3, g, be)


def kernel(x, mesh_senders, mesh_receivers, mesh_edge_feat,
           world_senders, world_receivers, world_edge_feat,
           me_W1, me_b1, me_W2, me_b2, me_W3, me_b3, me_g, me_be,
           we_W1, we_b1, we_W2, we_b2, we_W3, we_b3, we_g, we_be,
           nm_W1, nm_b1, nm_W2, nm_b2, nm_W3, nm_b3, nm_g, nm_be):
    em = mesh_senders.shape[0]
    ew = world_senders.shape[0]
    row = lambda v: v.reshape(1, D)

    w_sr = jnp.concatenate(
        [me_W1[:D], me_W1[D:2 * D], we_W1[:D], we_W1[D:2 * D]], axis=1)
    p, xa = _preproject(x, w_sr, nm_W1[:D])
    t_me_s = p[:, :D]
    t_me_r = p[:, D:2 * D]
    t_we_s = p[:, 2 * D:3 * D]
    t_we_r = p[:, 3 * D:]

    gs_m, gr_m = _gather_pair(t_me_s, t_me_r, mesh_senders, mesh_receivers, em)
    gs_w, gr_w = _gather_pair(t_we_s, t_we_r, world_senders, world_receivers, ew)

    new_mesh, proj_m = _edge_mlp(
        gs_m, gr_m, mesh_edge_feat, me_W1[2 * D:], row(me_b1), me_W2,
        row(me_b2), me_W3, row(me_b3), row(me_g), row(me_be),
        nm_W1[D:2 * D], em, 640)
    new_world, proj_w = _edge_mlp(
        gs_w, gr_w, world_edge_feat, we_W1[2 * D:], row(we_b1), we_W2,
        row(we_b2), we_W3, row(we_b3), row(we_g), row(we_be),
        nm_W1[2 * D:], ew, 640)

    parts = _scatter_partials(proj_m, mesh_receivers, proj_w, world_receivers,
                              em, ew)

    new_x = _node_mlp(x, xa, parts, row(nm_b1), nm_W2, row(nm_b2), nm_W3,
                      row(nm_b3), row(nm_g), row(nm_be))
    return (new_x, new_mesh, new_world)


# speedup vs baseline: 2.8192x; 2.8192x over previous
import jax
import jax.numpy as jnp
from jax import lax
from jax.experimental import pallas as pl
from jax.experimental.pallas import tpu as pltpu
from jax.experimental.pallas import tpu_sc as plsc

N_NODES = 10000
D = 128
NUM_CORES = 2
NUM_SUBCORES = 16
NW = NUM_CORES * NUM_SUBCORES
CH = 128


def _sc_mesh():
    return plsc.VectorSubcoreMesh(core_axis_name="c", subcore_axis_name="s")


def _preproject(x, w_sr, w_a):
    rb = 2000

    def body(x_ref, wsr_ref, wa_ref, p_ref, xa_ref):
        xb = x_ref[...]
        p_ref[...] = jnp.dot(xb, wsr_ref[...], preferred_element_type=jnp.float32)
        xa_ref[...] = jnp.dot(xb, wa_ref[...], preferred_element_type=jnp.float32)

    p, xa = pl.pallas_call(
        body,
        grid=(N_NODES // rb,),
        in_specs=[
            pl.BlockSpec((rb, D), lambda i: (i, 0)),
            pl.BlockSpec((D, 4 * D), lambda i: (0, 0)),
            pl.BlockSpec((D, D), lambda i: (0, 0)),
        ],
        out_specs=[
            pl.BlockSpec((rb, 4 * D), lambda i: (i, 0)),
            pl.BlockSpec((rb, D), lambda i: (i, 0)),
        ],
        out_shape=[
            jax.ShapeDtypeStruct((N_NODES, 4 * D), jnp.float32),
            jax.ShapeDtypeStruct((N_NODES, D), jnp.float32),
        ],
    )(x, w_sr, w_a)
    return p, xa


def _gather_pair(table_s, table_r, sidx, ridx, n_edges):
    per_w = n_edges // NW
    n_full, tail = divmod(per_w, CH)

    def body(ts_h, tr_h, si_h, ri_h, os_h, or_h,
             si_v, ri_v, bs_v, br_v, sit_v, rit_v, sem1, sem2):
        wid = lax.axis_index("s") * NUM_CORES + lax.axis_index("c")
        base = wid * per_w

        def chunk(i, carry):
            off = base + i * CH
            pltpu.sync_copy(si_h.at[pl.ds(off, CH)], si_v)
            pltpu.sync_copy(ri_h.at[pl.ds(off, CH)], ri_v)
            c1 = pltpu.async_copy(ts_h.at[si_v], bs_v, sem1)
            c2 = pltpu.async_copy(tr_h.at[ri_v], br_v, sem2)
            c1.wait()
            c2.wait()
            pltpu.sync_copy(bs_v, os_h.at[pl.ds(off, CH)])
            pltpu.sync_copy(br_v, or_h.at[pl.ds(off, CH)])
            return carry

        lax.fori_loop(0, n_full, chunk, None)
        if tail:
            off = base + n_full * CH
            pltpu.sync_copy(si_h.at[pl.ds(off, tail)], sit_v)
            pltpu.sync_copy(ri_h.at[pl.ds(off, tail)], rit_v)
            c1 = pltpu.async_copy(ts_h.at[sit_v], bs_v.at[pl.ds(0, tail)], sem1)
            c2 = pltpu.async_copy(tr_h.at[rit_v], br_v.at[pl.ds(0, tail)], sem2)
            c1.wait()
            c2.wait()
            pltpu.sync_copy(bs_v.at[pl.ds(0, tail)], os_h.at[pl.ds(off, tail)])
            pltpu.sync_copy(br_v.at[pl.ds(0, tail)], or_h.at[pl.ds(off, tail)])

    kern = pl.kernel(
        body,
        out_type=(
            jax.ShapeDtypeStruct((n_edges, D), jnp.float32),
            jax.ShapeDtypeStruct((n_edges, D), jnp.float32),
        ),
        mesh=_sc_mesh(),
        scratch_types=[
            pltpu.VMEM((CH,), jnp.int32),
            pltpu.VMEM((CH,), jnp.int32),
            pltpu.VMEM((CH, D), jnp.float32),
            pltpu.VMEM((CH, D), jnp.float32),
            pltpu.VMEM((max(tail, 8),), jnp.int32),
            pltpu.VMEM((max(tail, 8),), jnp.int32),
            pltpu.SemaphoreType.DMA,
            pltpu.SemaphoreType.DMA,
        ],
    )
    return kern(table_s, table_r, sidx, ridx)


def _edge_mlp(gs, gr, ef, w1c, b1, w2, b2, w3, b3, g, be, wproj, n_edges, eb):
    def body(gs_ref, gr_ref, ef_ref, w1c_ref, b1_ref, w2_ref, b2_ref,
             w3_ref, b3_ref, g_ref, be_ref, wp_ref, oe_ref, op_ref):
        ef_b = ef_ref[...]
        h = gs_ref[...] + gr_ref[...] + b1_ref[...] + jnp.dot(
            ef_b, w1c_ref[...], preferred_element_type=jnp.float32)
        h = jnp.maximum(h, 0.0)
        h = jnp.maximum(
            jnp.dot(h, w2_ref[...], preferred_element_type=jnp.float32)
            + b2_ref[...], 0.0)
        h = jnp.dot(h, w3_ref[...], preferred_element_type=jnp.float32) + b3_ref[...]
        mu = jnp.mean(h, axis=-1, keepdims=True)
        var = jnp.mean(jnp.square(h - mu), axis=-1, keepdims=True)
        y = (h - mu) / jnp.sqrt(var + 1e-5) * g_ref[...] + be_ref[...]
        oe_ref[...] = y + ef_b
        op_ref[...] = jnp.dot(y, wp_ref[...], preferred_element_type=jnp.float32)

    mat = lambda i: (0, 0)
    new_e, proj = pl.pallas_call(
        body,
        grid=(n_edges // eb,),
        in_specs=[
            pl.BlockSpec((eb, D), lambda i: (i, 0)),
            pl.BlockSpec((eb, D), lambda i: (i, 0)),
            pl.BlockSpec((eb, D), lambda i: (i, 0)),
            pl.BlockSpec((D, D), mat),
            pl.BlockSpec((1, D), mat),
            pl.BlockSpec((D, D), mat),
            pl.BlockSpec((1, D), mat),
            pl.BlockSpec((D, D), mat),
            pl.BlockSpec((1, D), mat),
            pl.BlockSpec((1, D), mat),
            pl.BlockSpec((1, D), mat),
            pl.BlockSpec((D, D), mat),
        ],
        out_specs=[
            pl.BlockSpec((eb, D), lambda i: (i, 0)),
            pl.BlockSpec((eb, D), lambda i: (i, 0)),
        ],
        out_shape=[
            jax.ShapeDtypeStruct((n_edges, D), jnp.float32),
            jax.ShapeDtypeStruct((n_edges, D), jnp.float32),
        ],
    )(gs, gr, ef, w1c, b1, w2, b2, w3, b3, g, be, wproj)
    return new_e, proj


def _scatter_partials(pm, im, pw, iw, em, ew):
    em_h = em // NUM_CORES
    ew_h = ew // NUM_CORES
    em_t = em_h // NUM_SUBCORES
    ew_t = ew_h // NUM_SUBCORES
    m_full, m_tail = divmod(em_t, CH)
    w_full, w_tail = divmod(ew_t, CH)
    rows_t = 640
    rows_last = N_NODES - (NUM_SUBCORES - 1) * rows_t

    def body(pm_h, im_h, pw_h, iw_h, zero_h, out_h,
             buf_v, idx_v, mt_v, wt_v, acc_s, sem):
        cid = lax.axis_index("c")
        sid = lax.axis_index("s")

        r0 = sid * rows_t

        @pl.when(sid < NUM_SUBCORES - 1)
        def _():
            pltpu.sync_copy(zero_h.at[pl.ds(r0, rows_t)],
                            acc_s.at[pl.ds(r0, rows_t)])

        @pl.when(sid == NUM_SUBCORES - 1)
        def _():
            pltpu.sync_copy(zero_h.at[pl.ds(r0, rows_last)],
                            acc_s.at[pl.ds(r0, rows_last)])

        plsc.subcore_barrier()

        def scatter_chunks(p_h, i_h, base, nfull, tail, tidx_v):
            def chunk(i, carry):
                off = base + i * CH
                pltpu.sync_copy(i_h.at[pl.ds(off, CH)], idx_v)
                pltpu.sync_copy(p_h.at[pl.ds(off, CH)], buf_v)
                pltpu.sync_copy(buf_v, acc_s.at[idx_v], add=True)
                return carry

            lax.fori_loop(0, nfull, chunk, None)
            if tail:
                off = base + nfull * CH
                pltpu.sync_copy(i_h.at[pl.ds(off, tail)], tidx_v)
                pltpu.sync_copy(p_h.at[pl.ds(off, tail)], buf_v.at[pl.ds(0, tail)])
                pltpu.sync_copy(buf_v.at[pl.ds(0, tail)], acc_s.at[tidx_v], add=True)

        scatter_chunks(pm_h, im_h, cid * em_h + sid * em_t, m_full, m_tail, mt_v)
        scatter_chunks(pw_h, iw_h, cid * ew_h + sid * ew_t, w_full, w_tail, wt_v)
        plsc.subcore_barrier()

        @pl.when(sid < NUM_SUBCORES - 1)
        def _():
            pltpu.sync_copy(acc_s.at[pl.ds(r0, rows_t)],
                            out_h.at[cid].at[pl.ds(r0, rows_t)])

        @pl.when(sid == NUM_SUBCORES - 1)
        def _():
            pltpu.sync_copy(acc_s.at[pl.ds(r0, rows_last)],
                            out_h.at[cid].at[pl.ds(r0, rows_last)])

    kern = pl.kernel(
        body,
        out_type=jax.ShapeDtypeStruct((NUM_CORES, N_NODES, D), jnp.float32),
        mesh=_sc_mesh(),
        scratch_types=[
            pltpu.VMEM((CH, D), jnp.float32),
            pltpu.VMEM((CH,), jnp.int32),
            pltpu.VMEM((max(m_tail, 8),), jnp.int32),
            pltpu.VMEM((max(w_tail, 8),), jnp.int32),
            pltpu.VMEM_SHARED((N_NODES, D), jnp.float32),
            pltpu.SemaphoreType.DMA,
        ],
    )
    zero = jnp.zeros((N_NODES, D), jnp.float32)
    return kern(pm, im, pw, iw, zero)


def _node_mlp(x, xa, parts, b1, w2, b2, w3, b3, g, be):
    rb = 2000

    def body(x_ref, xa_ref, p0_ref, p1_ref, b1_ref, w2_ref, b2_ref,
             w3_ref, b3_ref, g_ref, be_ref, o_ref):
        h = xa_ref[...] + p0_ref[0] + p1_ref[0] + b1_ref[...]
        h = jnp.maximum(h, 0.0)
        h = jnp.maximum(
            jnp.dot(h, w2_ref[...], preferred_element_type=jnp.float32)
            + b2_ref[...], 0.0)
        h = jnp.dot(h, w3_ref[...], preferred_element_type=jnp.float32) + b3_ref[...]
        mu = jnp.mean(h, axis=-1, keepdims=True)
        var = jnp.mean(jnp.square(h - mu), axis=-1, keepdims=True)
        y = (h - mu) / jnp.sqrt(var + 1e-5) * g_ref[...] + be_ref[...]
        o_ref[...] = y + x_ref[...]

    mat = lambda i: (0, 0)
    return pl.pallas_call(
        body,
        grid=(N_NODES // rb,),
        in_specs=[
            pl.BlockSpec((rb, D), lambda i: (i, 0)),
            pl.BlockSpec((rb, D), lambda i: (i, 0)),
            pl.BlockSpec((1, rb, D), lambda i: (0, i, 0)),
            pl.BlockSpec((1, rb, D), lambda i: (1, i, 0)),
            pl.BlockSpec((1, D), mat),
            pl.BlockSpec((D, D), mat),
            pl.BlockSpec((1, D), mat),
            pl.BlockSpec((D, D), mat),
            pl.BlockSpec((1, D), mat),
            pl.BlockSpec((1, D), mat),
            pl.BlockSpec((1, D), mat),
        ],
        out_specs=pl.BlockSpec((rb, D), lambda i: (i, 0)),
        out_shape=jax.ShapeDtypeStruct((N_NODES, D), jnp.float32),
    )(x, xa, parts, parts, b1, w2, b2, w3, b3, g, be)


def kernel(x, mesh_senders, mesh_receivers, mesh_edge_feat,
           world_senders, world_receivers, world_edge_feat,
           me_W1, me_b1, me_W2, me_b2, me_W3, me_b3, me_g, me_be,
           we_W1, we_b1, we_W2, we_b2, we_W3, we_b3, we_g, we_be,
           nm_W1, nm_b1, nm_W2, nm_b2, nm_W3, nm_b3, nm_g, nm_be):
    em = mesh_senders.shape[0]
    ew = world_senders.shape[0]
    row = lambda v: v.reshape(1, D)

    w_sr = jnp.concatenate(
        [me_W1[:D], me_W1[D:2 * D], we_W1[:D], we_W1[D:2 * D]], axis=1)
    p, xa = _preproject(x, w_sr, nm_W1[:D])
    t_me_s = p[:, :D]
    t_me_r = p[:, D:2 * D]
    t_we_s = p[:, 2 * D:3 * D]
    t_we_r = p[:, 3 * D:]

    gs_m, gr_m = _gather_pair(t_me_s, t_me_r, mesh_senders, mesh_receivers, em)
    gs_w, gr_w = _gather_pair(t_we_s, t_we_r, world_senders, world_receivers, ew)

    new_mesh, proj_m = _edge_mlp(
        gs_m, gr_m, mesh_edge_feat, me_W1[2 * D:], row(me_b1), me_W2,
        row(me_b2), me_W3, row(me_b3), row(me_g), row(me_be),
        nm_W1[D:2 * D], em, 640)
    new_world, proj_w = _edge_mlp(
        gs_w, gr_w, world_edge_feat, we_W1[2 * D:], row(we_b1), we_W2,
        row(we_b2), we_W3, row(we_b3), row(we_g), row(we_be),
        nm_W1[2 * D:], ew, 640)

    parts = _scatter_partials(proj_m, mesh_receivers, proj_w, world_receivers,
                              em, ew)

    new_x = _node_mlp(x, xa, parts, row(nm_b1), nm_W2, row(nm_b2), nm_W3,
                      row(nm_b3), row(nm_g), row(nm_be))
    return (new_x, new_mesh, new_world)


# speedup vs baseline: 2.8293x; 1.0036x over previous
import jax
import jax.numpy as jnp
from jax import lax
from jax.experimental import pallas as pl
from jax.experimental.pallas import tpu as pltpu
from jax.experimental.pallas import tpu_sc as plsc

N_NODES = 10000
D = 128
NUM_CORES = 2
NUM_SUBCORES = 16
NW = NUM_CORES * NUM_SUBCORES
CH = 128


def _sc_mesh():
    return plsc.VectorSubcoreMesh(core_axis_name="c", subcore_axis_name="s")


def _preproject(x, w_sr, w_a):
    rb = 2000

    def body(x_ref, wsr_ref, wa_ref, t0_ref, t1_ref, t2_ref, t3_ref, xa_ref):
        xb = x_ref[...]
        p = jnp.dot(xb, wsr_ref[...], preferred_element_type=jnp.float32)
        t0_ref[...] = p[:, :D]
        t1_ref[...] = p[:, D:2 * D]
        t2_ref[...] = p[:, 2 * D:3 * D]
        t3_ref[...] = p[:, 3 * D:]
        xa_ref[...] = jnp.dot(xb, wa_ref[...], preferred_element_type=jnp.float32)

    blk = pl.BlockSpec((rb, D), lambda i: (i, 0))
    outs = pl.pallas_call(
        body,
        grid=(N_NODES // rb,),
        in_specs=[
            pl.BlockSpec((rb, D), lambda i: (i, 0)),
            pl.BlockSpec((D, 4 * D), lambda i: (0, 0)),
            pl.BlockSpec((D, D), lambda i: (0, 0)),
        ],
        out_specs=[blk, blk, blk, blk, blk],
        out_shape=[jax.ShapeDtypeStruct((N_NODES, D), jnp.float32)] * 5,
    )(x, w_sr, w_a)
    return outs


def _gather_pair(table_s, table_r, sidx, ridx, n_edges):
    n_chunks = n_edges // CH
    n_base, n_rem = divmod(n_chunks, NW)

    def body(ts_h, tr_h, si_h, ri_h, os_h, or_h,
             si_v, ri_v, bs_v, br_v, sem1, sem2):
        wid = lax.axis_index("s") * NUM_CORES + lax.axis_index("c")
        n_mine = n_base + (wid < n_rem).astype(jnp.int32)

        def chunk(i, carry):
            off = (wid + i * NW) * CH
            pltpu.sync_copy(si_h.at[pl.ds(off, CH)], si_v)
            pltpu.sync_copy(ri_h.at[pl.ds(off, CH)], ri_v)
            c1 = pltpu.async_copy(ts_h.at[si_v], bs_v, sem1)
            c2 = pltpu.async_copy(tr_h.at[ri_v], br_v, sem2)
            c1.wait()
            c2.wait()
            pltpu.sync_copy(bs_v, os_h.at[pl.ds(off, CH)])
            pltpu.sync_copy(br_v, or_h.at[pl.ds(off, CH)])
            return carry

        lax.fori_loop(0, n_mine, chunk, None)

    kern = pl.kernel(
        body,
        out_type=(
            jax.ShapeDtypeStruct((n_edges, D), jnp.float32),
            jax.ShapeDtypeStruct((n_edges, D), jnp.float32),
        ),
        mesh=_sc_mesh(),
        scratch_types=[
            pltpu.VMEM((CH,), jnp.int32),
            pltpu.VMEM((CH,), jnp.int32),
            pltpu.VMEM((CH, D), jnp.float32),
            pltpu.VMEM((CH, D), jnp.float32),
            pltpu.SemaphoreType.DMA,
            pltpu.SemaphoreType.DMA,
        ],
    )
    return kern(table_s, table_r, sidx, ridx)


def _edge_mlp(gs, gr, ef, w1c, b1, w2, b2, w3, b3, g, be, wproj, n_edges, eb):
    def body(gs_ref, gr_ref, ef_ref, w1c_ref, b1_ref, w2_ref, b2_ref,
             w3_ref, b3_ref, g_ref, be_ref, wp_ref, oe_ref, op_ref):
        ef_b = ef_ref[...]
        h = (gs_ref[...] + gr_ref[...] + b1_ref[...]
             + jnp.dot(ef_b.astype(jnp.bfloat16), w1c_ref[...],
                       preferred_element_type=jnp.float32))
        h = jnp.maximum(h, 0.0)
        h = jnp.maximum(
            jnp.dot(h.astype(jnp.bfloat16), w2_ref[...],
                    preferred_element_type=jnp.float32) + b2_ref[...], 0.0)
        h = jnp.dot(h.astype(jnp.bfloat16), w3_ref[...],
                    preferred_element_type=jnp.float32) + b3_ref[...]
        mu = jnp.mean(h, axis=-1, keepdims=True)
        var = jnp.mean(jnp.square(h - mu), axis=-1, keepdims=True)
        y = (h - mu) / jnp.sqrt(var + 1e-5) * g_ref[...] + be_ref[...]
        oe_ref[...] = y + ef_b
        op_ref[...] = jnp.dot(y.astype(jnp.bfloat16), wp_ref[...],
                              preferred_element_type=jnp.float32)

    mat = lambda i: (0, 0)
    new_e, proj = pl.pallas_call(
        body,
        grid=(n_edges // eb,),
        in_specs=[
            pl.BlockSpec((eb, D), lambda i: (i, 0)),
            pl.BlockSpec((eb, D), lambda i: (i, 0)),
            pl.BlockSpec((eb, D), lambda i: (i, 0)),
            pl.BlockSpec((D, D), mat),
            pl.BlockSpec((1, D), mat),
            pl.BlockSpec((D, D), mat),
            pl.BlockSpec((1, D), mat),
            pl.BlockSpec((D, D), mat),
            pl.BlockSpec((1, D), mat),
            pl.BlockSpec((1, D), mat),
            pl.BlockSpec((1, D), mat),
            pl.BlockSpec((D, D), mat),
        ],
        out_specs=[
            pl.BlockSpec((eb, D), lambda i: (i, 0)),
            pl.BlockSpec((eb, D), lambda i: (i, 0)),
        ],
        out_shape=[
            jax.ShapeDtypeStruct((n_edges, D), jnp.float32),
            jax.ShapeDtypeStruct((n_edges, D), jnp.float32),
        ],
    )(gs, gr, ef, w1c, b1, w2, b2, w3, b3, g, be, wproj)
    return new_e, proj


def _scatter_partials(pm, im, pw, iw, em, ew):
    em_h = em // NUM_CORES
    ew_h = ew // NUM_CORES
    em_t = em_h // NUM_SUBCORES
    ew_t = ew_h // NUM_SUBCORES
    m_full, m_tail = divmod(em_t, CH)
    w_full, w_tail = divmod(ew_t, CH)
    rows_t = 640
    rows_last = N_NODES - (NUM_SUBCORES - 1) * rows_t

    def body(pm_h, im_h, pw_h, iw_h, zero_h, out_h,
             buf_v, idx_v, mt_v, wt_v, acc_s, sem):
        cid = lax.axis_index("c")
        sid = lax.axis_index("s")

        r0 = sid * rows_t

        @pl.when(sid < NUM_SUBCORES - 1)
        def _():
            pltpu.sync_copy(zero_h.at[pl.ds(r0, rows_t)],
                            acc_s.at[pl.ds(r0, rows_t)])

        @pl.when(sid == NUM_SUBCORES - 1)
        def _():
            pltpu.sync_copy(zero_h.at[pl.ds(r0, rows_last)],
                            acc_s.at[pl.ds(r0, rows_last)])

        plsc.subcore_barrier()

        def scatter_chunks(p_h, i_h, base, nfull, tail, tidx_v):
            def chunk(i, carry):
                off = base + i * CH
                pltpu.sync_copy(i_h.at[pl.ds(off, CH)], idx_v)
                pltpu.sync_copy(p_h.at[pl.ds(off, CH)], buf_v)
                pltpu.sync_copy(buf_v, acc_s.at[idx_v], add=True)
                return carry

            lax.fori_loop(0, nfull, chunk, None)
            if tail:
                off = base + nfull * CH
                pltpu.sync_copy(i_h.at[pl.ds(off, tail)], tidx_v)
                pltpu.sync_copy(p_h.at[pl.ds(off, tail)], buf_v.at[pl.ds(0, tail)])
                pltpu.sync_copy(buf_v.at[pl.ds(0, tail)], acc_s.at[tidx_v], add=True)

        scatter_chunks(pm_h, im_h, cid * em_h + sid * em_t, m_full, m_tail, mt_v)
        scatter_chunks(pw_h, iw_h, cid * ew_h + sid * ew_t, w_full, w_tail, wt_v)
        plsc.subcore_barrier()

        @pl.when(sid < NUM_SUBCORES - 1)
        def _():
            pltpu.sync_copy(acc_s.at[pl.ds(r0, rows_t)],
                            out_h.at[cid].at[pl.ds(r0, rows_t)])

        @pl.when(sid == NUM_SUBCORES - 1)
        def _():
            pltpu.sync_copy(acc_s.at[pl.ds(r0, rows_last)],
                            out_h.at[cid].at[pl.ds(r0, rows_last)])

    kern = pl.kernel(
        body,
        out_type=jax.ShapeDtypeStruct((NUM_CORES, N_NODES, D), jnp.float32),
        mesh=_sc_mesh(),
        scratch_types=[
            pltpu.VMEM((CH, D), jnp.float32),
            pltpu.VMEM((CH,), jnp.int32),
            pltpu.VMEM((max(m_tail, 8),), jnp.int32),
            pltpu.VMEM((max(w_tail, 8),), jnp.int32),
            pltpu.VMEM_SHARED((N_NODES, D), jnp.float32),
            pltpu.SemaphoreType.DMA,
        ],
    )
    zero = jnp.zeros((N_NODES, D), jnp.float32)
    return kern(pm, im, pw, iw, zero)


def _node_mlp(x, xa, parts, b1, w2, b2, w3, b3, g, be):
    rb = 2000

    def body(x_ref, xa_ref, p0_ref, p1_ref, b1_ref, w2_ref, b2_ref,
             w3_ref, b3_ref, g_ref, be_ref, o_ref):
        h = xa_ref[...] + p0_ref[0] + p1_ref[0] + b1_ref[...]
        h = jnp.maximum(h, 0.0)
        h = jnp.maximum(
            jnp.dot(h, w2_ref[...], preferred_element_type=jnp.float32)
            + b2_ref[...], 0.0)
        h = jnp.dot(h, w3_ref[...], preferred_element_type=jnp.float32) + b3_ref[...]
        mu = jnp.mean(h, axis=-1, keepdims=True)
        var = jnp.mean(jnp.square(h - mu), axis=-1, keepdims=True)
        y = (h - mu) / jnp.sqrt(var + 1e-5) * g_ref[...] + be_ref[...]
        o_ref[...] = y + x_ref[...]

    mat = lambda i: (0, 0)
    return pl.pallas_call(
        body,
        grid=(N_NODES // rb,),
        in_specs=[
            pl.BlockSpec((rb, D), lambda i: (i, 0)),
            pl.BlockSpec((rb, D), lambda i: (i, 0)),
            pl.BlockSpec((1, rb, D), lambda i: (0, i, 0)),
            pl.BlockSpec((1, rb, D), lambda i: (1, i, 0)),
            pl.BlockSpec((1, D), mat),
            pl.BlockSpec((D, D), mat),
            pl.BlockSpec((1, D), mat),
            pl.BlockSpec((D, D), mat),
            pl.BlockSpec((1, D), mat),
            pl.BlockSpec((1, D), mat),
            pl.BlockSpec((1, D), mat),
        ],
        out_specs=pl.BlockSpec((rb, D), lambda i: (i, 0)),
        out_shape=jax.ShapeDtypeStruct((N_NODES, D), jnp.float32),
    )(x, xa, parts, parts, b1, w2, b2, w3, b3, g, be)


def kernel(x, mesh_senders, mesh_receivers, mesh_edge_feat,
           world_senders, world_receivers, world_edge_feat,
           me_W1, me_b1, me_W2, me_b2, me_W3, me_b3, me_g, me_be,
           we_W1, we_b1, we_W2, we_b2, we_W3, we_b3, we_g, we_be,
           nm_W1, nm_b1, nm_W2, nm_b2, nm_W3, nm_b3, nm_g, nm_be):
    em = mesh_senders.shape[0]
    ew = world_senders.shape[0]
    row = lambda v: v.reshape(1, D)

    bf = lambda w: w.astype(jnp.bfloat16)
    w_sr = jnp.concatenate(
        [me_W1[:D], me_W1[D:2 * D], we_W1[:D], we_W1[D:2 * D]], axis=1)
    t_me_s, t_me_r, t_we_s, t_we_r, xa = _preproject(x, w_sr, nm_W1[:D])

    gs_m, gr_m = _gather_pair(t_me_s, t_me_r, mesh_senders, mesh_receivers, em)
    gs_w, gr_w = _gather_pair(t_we_s, t_we_r, world_senders, world_receivers, ew)

    new_mesh, proj_m = _edge_mlp(
        gs_m, gr_m, mesh_edge_feat, bf(me_W1[2 * D:]), row(me_b1), bf(me_W2),
        row(me_b2), bf(me_W3), row(me_b3), row(me_g), row(me_be),
        bf(nm_W1[D:2 * D]), em, 640)
    new_world, proj_w = _edge_mlp(
        gs_w, gr_w, world_edge_feat, bf(we_W1[2 * D:]), row(we_b1), bf(we_W2),
        row(we_b2), bf(we_W3), row(we_b3), row(we_g), row(we_be),
        bf(nm_W1[2 * D:]), ew, 640)

    parts = _scatter_partials(proj_m, mesh_receivers, proj_w, world_receivers,
                              em, ew)

    new_x = _node_mlp(x, xa, parts, row(nm_b1), nm_W2, row(nm_b2), nm_W3,
                      row(nm_b3), row(nm_g), row(nm_be))
    return (new_x, new_mesh, new_world)
